# Initial kernel scaffold; baseline (speedup 1.0000x reference)
import jax
import jax.numpy as jnp
from jax.experimental import pallas as pl


def kernel(sub, rel, edge_index, edge_type, init_embed, init_rel, W1, W1_rel, W2, W2_rel):
    raise NotImplementedError("write your pallas kernel here")


# speedup vs baseline: 2.7118x; 2.7118x over previous
import functools

import jax
import jax.numpy as jnp
from jax import lax
from jax.experimental import pallas as pl
from jax.experimental.pallas import tpu as pltpu
from jax.experimental.pallas import tpu_sc as plsc

NUM_ENT = 10000
NUM_RELROWS = 400
N_EDGES = 160000
D = 256
H = 128
NS = 16
W = 80
EDGES_PER_TILE = N_EDGES // NS
WINDOWS = EDGES_PER_TILE // W
ACC_SLAB = 640
ZROWS = 80
DEG_PAD = 10240
TEMPERATURE = 0.5
BATCH = 1024


def _sc_agg_body(with_deg, x_lo, x_hi, rel_lo, rel_hi, src_h, dst_h, et_h,
                 *refs):
    if with_deg:
        out_lo, out_hi, deg_out = refs[:3]
        scratch = refs[3:]
    else:
        out_lo, out_hi = refs[:2]
        scratch = refs[2:]
    (xbuf, relbuf, zbuf, src_v, dst_v, et_v, ones_v, zdeg_v, rel_sh, acc_sh,
     deg_sh, sem) = scratch

    cid = lax.axis_index("c")
    sid = lax.axis_index("s")

    @pl.when(sid < NUM_RELROWS // ZROWS)
    def _():
        o = pl.multiple_of(sid * ZROWS, 8)

        @pl.when(cid == 0)
        def _():
            pltpu.sync_copy(rel_lo.at[pl.ds(o, ZROWS)], zbuf)

        @pl.when(cid == 1)
        def _():
            pltpu.sync_copy(rel_hi.at[pl.ds(o, ZROWS)], zbuf)

        pltpu.sync_copy(zbuf, rel_sh.at[pl.ds(o, ZROWS)])

    def _zrow(i, _):
        for j in range(H // 16):
            zbuf[i, pl.ds(j * 16, 16)] = jnp.zeros((16,), jnp.float32)
        return 0
    lax.fori_loop(0, ZROWS, _zrow, 0)
    for k in range(W // 16):
        ones_v[pl.ds(k * 16, 16)] = jnp.ones((16,), jnp.float32)
    if with_deg:
        def _zdeg(i, _):
            zdeg_v[pl.ds(i * 16, 16)] = jnp.zeros((16,), jnp.float32)
            return 0
        lax.fori_loop(0, 1024 // 16, _zdeg, 0)

    for k in range(ACC_SLAB // ZROWS):
        off = sid * ACC_SLAB + k * ZROWS

        @pl.when(off < NUM_ENT)
        def _():
            pltpu.sync_copy(zbuf,
                            acc_sh.at[pl.ds(pl.multiple_of(off, 8), ZROWS)])
    if with_deg:
        @pl.when((cid == 0) & (sid < 10))
        def _():
            pltpu.sync_copy(
                zdeg_v, deg_sh.at[pl.ds(pl.multiple_of(sid * 1024, 8), 1024)])

    plsc.subcore_barrier()

    def _window(i, _):
        base = pl.multiple_of(sid * EDGES_PER_TILE + i * W, 8)
        pltpu.sync_copy(src_h.at[pl.ds(base, W)], src_v)
        pltpu.sync_copy(dst_h.at[pl.ds(base, W)], dst_v)
        pltpu.sync_copy(et_h.at[pl.ds(base, W)], et_v)

        @pl.when(cid == 0)
        def _():
            pltpu.async_copy(x_lo.at[src_v], xbuf, sem).wait()

        @pl.when(cid == 1)
        def _():
            pltpu.async_copy(x_hi.at[src_v], xbuf, sem).wait()

        pltpu.async_copy(rel_sh.at[et_v], relbuf, sem).wait()

        def _edge(e, _):
            for j in range(H // 16):
                xv = xbuf[e, pl.ds(j * 16, 16)]
                rv = relbuf[e, pl.ds(j * 16, 16)]
                xbuf[e, pl.ds(j * 16, 16)] = xv * rv
            return 0
        lax.fori_loop(0, W, _edge, 0)

        pltpu.sync_copy(xbuf, acc_sh.at[dst_v], add=True)
        if with_deg:
            @pl.when(cid == 0)
            def _():
                pltpu.sync_copy(ones_v, deg_sh.at[dst_v], add=True)
        return 0

    lax.fori_loop(0, WINDOWS, _window, 0)

    plsc.subcore_barrier()

    def _copy_out(out_ref):
        for k in range(ACC_SLAB // ZROWS):
            off = sid * ACC_SLAB + k * ZROWS

            @pl.when(off < NUM_ENT)
            def _():
                o = pl.multiple_of(off, 8)
                pltpu.sync_copy(acc_sh.at[pl.ds(o, ZROWS)], zbuf)
                pltpu.sync_copy(zbuf, out_ref.at[pl.ds(o, ZROWS)])

    @pl.when(cid == 0)
    def _():
        _copy_out(out_lo)

    @pl.when(cid == 1)
    def _():
        _copy_out(out_hi)

    if with_deg:
        @pl.when((cid == 0) & (sid < 10))
        def _():
            off = pl.multiple_of(sid * 1000, 8)
            pltpu.sync_copy(deg_sh.at[pl.ds(off, 1000)],
                            zdeg_v.at[pl.ds(0, 1000)])
            pltpu.sync_copy(zdeg_v.at[pl.ds(0, 1000)],
                            deg_out.at[pl.ds(off, 1000)])


def _make_sc_agg(with_deg):
    out_type = [
        jax.ShapeDtypeStruct((NUM_ENT, H), jnp.float32),
        jax.ShapeDtypeStruct((NUM_ENT, H), jnp.float32),
    ]
    if with_deg:
        out_type.append(jax.ShapeDtypeStruct((NUM_ENT,), jnp.float32))
    mesh = plsc.VectorSubcoreMesh(core_axis_name="c", subcore_axis_name="s")
    return pl.kernel(
        functools.partial(_sc_agg_body, with_deg),
        out_type=out_type,
        mesh=mesh,
        scratch_types=[
            pltpu.VMEM((W, H), jnp.float32),
            pltpu.VMEM((W, H), jnp.float32),
            pltpu.VMEM((ZROWS, H), jnp.float32),
            pltpu.VMEM((W,), jnp.int32),
            pltpu.VMEM((W,), jnp.int32),
            pltpu.VMEM((W,), jnp.int32),
            pltpu.VMEM((W,), jnp.float32),
            pltpu.VMEM((1024,), jnp.float32),
            pltpu.VMEM_SHARED((NUM_RELROWS, H), jnp.float32),
            pltpu.VMEM_SHARED((NUM_ENT, H), jnp.float32),
            pltpu.VMEM_SHARED((DEG_PAD,), jnp.float32),
            pltpu.SemaphoreType.DMA,
        ],
        compiler_params=pltpu.CompilerParams(needs_layout_passes=False),
        name="sc_agg_deg" if with_deg else "sc_agg",
    )


_sc_agg_deg = _make_sc_agg(True)


def _sc_lookup_body(x_h, r_h, sub_h, rel_h, sub_out, rel_out,
                    idx_v, xrows_v, rrows_v, sem):
    cid = lax.axis_index("c")
    sid = lax.axis_index("s")
    wid = sid * 2 + cid
    bpw = BATCH // 32
    base = pl.multiple_of(wid * bpw, 8)
    pltpu.sync_copy(sub_h.at[pl.ds(base, bpw)], idx_v)
    pltpu.async_copy(x_h.at[idx_v], xrows_v, sem).wait()
    pltpu.sync_copy(xrows_v, sub_out.at[pl.ds(base, bpw)])
    pltpu.sync_copy(rel_h.at[pl.ds(base, bpw)], idx_v)
    pltpu.async_copy(r_h.at[idx_v], rrows_v, sem).wait()
    pltpu.sync_copy(rrows_v, rel_out.at[pl.ds(base, bpw)])


_sc_lookup = pl.kernel(
    _sc_lookup_body,
    out_type=[
        jax.ShapeDtypeStruct((BATCH, D), jnp.float32),
        jax.ShapeDtypeStruct((BATCH, D), jnp.float32),
    ],
    mesh=plsc.VectorSubcoreMesh(core_axis_name="c", subcore_axis_name="s"),
    scratch_types=[
        pltpu.VMEM((BATCH // 32,), jnp.int32),
        pltpu.VMEM((BATCH // 32, D), jnp.float32),
        pltpu.VMEM((BATCH // 32, D), jnp.float32),
        pltpu.SemaphoreType.DMA,
    ],
    name="sc_lookup",
)


RB = 1000


def _tc1_body(agg_lo, agg_hi, deg, W1, r0, W1r,
              x1lo, x1hi, r1lo, r1hi):
    i = pl.program_id(0)
    norm = 1.0 / jnp.maximum(deg[...], 1.0)
    a_lo = agg_lo[...] * norm
    a_hi = agg_hi[...] * norm
    w = W1[...]
    h = (jnp.dot(a_lo, w[:H, :], preferred_element_type=jnp.float32)
         + jnp.dot(a_hi, w[H:, :], preferred_element_type=jnp.float32))
    x1 = jnp.tanh(h)
    x1lo[...] = x1[:, :H]
    x1hi[...] = x1[:, H:]

    @pl.when(i == 0)
    def _():
        r1 = jnp.dot(r0[...], W1r[...], preferred_element_type=jnp.float32)
        r1lo[...] = r1[:, :H]
        r1hi[...] = r1[:, H:]


_tc1 = pl.pallas_call(
    _tc1_body,
    grid=(NUM_ENT // RB,),
    in_specs=[
        pl.BlockSpec((RB, H), lambda i: (i, 0)),
        pl.BlockSpec((RB, H), lambda i: (i, 0)),
        pl.BlockSpec((RB, 1), lambda i: (i, 0)),
        pl.BlockSpec((D, D), lambda i: (0, 0)),
        pl.BlockSpec((NUM_RELROWS, D), lambda i: (0, 0)),
        pl.BlockSpec((D, D), lambda i: (0, 0)),
    ],
    out_specs=[
        pl.BlockSpec((RB, H), lambda i: (i, 0)),
        pl.BlockSpec((RB, H), lambda i: (i, 0)),
        pl.BlockSpec((NUM_RELROWS, H), lambda i: (0, 0)),
        pl.BlockSpec((NUM_RELROWS, H), lambda i: (0, 0)),
    ],
    out_shape=[
        jax.ShapeDtypeStruct((NUM_ENT, H), jnp.float32),
        jax.ShapeDtypeStruct((NUM_ENT, H), jnp.float32),
        jax.ShapeDtypeStruct((NUM_RELROWS, H), jnp.float32),
        jax.ShapeDtypeStruct((NUM_RELROWS, H), jnp.float32),
    ],
    name="tc_dense1",
)


def _tc2_body(agg_lo, agg_hi, deg, W2, r1lo, r1hi, W2r,
              x2, r2_out, cor_out):
    i = pl.program_id(0)
    norm = 1.0 / jnp.maximum(deg[...], 1.0)
    a_lo = agg_lo[...] * norm
    a_hi = agg_hi[...] * norm
    w = W2[...]
    h = (jnp.dot(a_lo, w[:H, :], preferred_element_type=jnp.float32)
         + jnp.dot(a_hi, w[H:, :], preferred_element_type=jnp.float32))
    x2[...] = jnp.tanh(h)

    @pl.when(i == 0)
    def _():
        wr = W2r[...]
        r2 = (jnp.dot(r1lo[...], wr[:H, :], preferred_element_type=jnp.float32)
              + jnp.dot(r1hi[...], wr[H:, :], preferred_element_type=jnp.float32))
        r2_out[...] = r2
        nrm = jnp.sqrt(jnp.sum(r2 * r2, axis=1, keepdims=True))
        nr = r2 / nrm
        pos_smi = jnp.sum(nr * nr, axis=1)
        ttl_smi = lax.dot_general(nr, nr, (((1,), (1,)), ((), ())),
                                  preferred_element_type=jnp.float32)
        pos_scores = jnp.exp(pos_smi / TEMPERATURE)
        ttl_scores = jnp.exp(ttl_smi / TEMPERATURE)
        semi = jnp.exp(jnp.float32(1.0) / TEMPERATURE)
        ttl = jnp.sum(ttl_scores, axis=1) - semi + pos_scores
        cor = -jnp.sum(jnp.log(pos_scores / ttl))
        cor_out[...] = cor.reshape(1, 1)


_tc2 = pl.pallas_call(
    _tc2_body,
    grid=(NUM_ENT // RB,),
    in_specs=[
        pl.BlockSpec((RB, H), lambda i: (i, 0)),
        pl.BlockSpec((RB, H), lambda i: (i, 0)),
        pl.BlockSpec((RB, 1), lambda i: (i, 0)),
        pl.BlockSpec((D, D), lambda i: (0, 0)),
        pl.BlockSpec((NUM_RELROWS, H), lambda i: (0, 0)),
        pl.BlockSpec((NUM_RELROWS, H), lambda i: (0, 0)),
        pl.BlockSpec((D, D), lambda i: (0, 0)),
    ],
    out_specs=[
        pl.BlockSpec((RB, D), lambda i: (i, 0)),
        pl.BlockSpec((NUM_RELROWS, D), lambda i: (0, 0)),
        pl.BlockSpec((1, 1), lambda i: (0, 0)),
    ],
    out_shape=[
        jax.ShapeDtypeStruct((NUM_ENT, D), jnp.float32),
        jax.ShapeDtypeStruct((NUM_RELROWS, D), jnp.float32),
        jax.ShapeDtypeStruct((1, 1), jnp.float32),
    ],
    name="tc_dense2",
)


def kernel(sub, rel, edge_index, edge_type, init_embed, init_rel,
           W1, W1_rel, W2, W2_rel):
    sub = sub.astype(jnp.int32)
    rel = rel.astype(jnp.int32)
    src = edge_index[0].astype(jnp.int32)
    dst = edge_index[1].astype(jnp.int32)
    et = edge_type.astype(jnp.int32)

    x0_lo = init_embed[:, :H]
    x0_hi = init_embed[:, H:]
    r0_lo = init_rel[:, :H]
    r0_hi = init_rel[:, H:]

    agg1_lo, agg1_hi, deg = _sc_agg_deg(x0_lo, x0_hi, r0_lo, r0_hi,
                                        src, dst, et)
    deg2d = deg.reshape(NUM_ENT, 1)
    x1lo, x1hi, r1lo, r1hi = _tc1(agg1_lo, agg1_hi, deg2d, W1,
                                  init_rel, W1_rel)
    agg2_lo, agg2_hi, _deg2 = _sc_agg_deg(x1lo, x1hi, r1lo, r1hi,
                                           src, dst, et)
    x2, r2, cor = _tc2(agg2_lo, agg2_hi, deg2d, W2, r1lo, r1hi, W2_rel)
    sub_emb, rel_emb = _sc_lookup(x2, r2, sub, rel)
    return sub_emb, rel_emb, x2, cor[0, 0]


# speedup vs baseline: 2.7404x; 1.0106x over previous
import functools

import jax
import jax.numpy as jnp
from jax import lax
from jax.experimental import pallas as pl
from jax.experimental.pallas import tpu as pltpu
from jax.experimental.pallas import tpu_sc as plsc

NUM_ENT = 10000
NUM_RELROWS = 400
N_EDGES = 160000
D = 256
H = 128
NS = 16
W = 80
EDGES_PER_TILE = N_EDGES // NS
WINDOWS = EDGES_PER_TILE // W
NWB = 5
NBATCH = WINDOWS // NWB
ACC_SLAB = 640
ZROWS = 80
DEG_PAD = 10240
TEMPERATURE = 0.5
BATCH = 1024


def _sc_agg_body(with_deg, x_lo, x_hi, rel_lo, rel_hi, src_h, dst_h, et_h,
                 *refs):
    if with_deg:
        out_lo, out_hi, deg_out = refs[:3]
        scratch = refs[3:]
    else:
        out_lo, out_hi = refs[:2]
        scratch = refs[2:]
    (xbuf, relbuf, zbuf, src_v, dst_v, et_v, ones_v,
     zdeg_v, rel_sh, acc_sh, deg_sh, sem) = scratch

    cid = lax.axis_index("c")
    sid = lax.axis_index("s")

    @pl.when(sid < NUM_RELROWS // ZROWS)
    def _():
        o = pl.multiple_of(sid * ZROWS, 8)

        @pl.when(cid == 0)
        def _():
            pltpu.sync_copy(rel_lo.at[pl.ds(o, ZROWS)], zbuf)

        @pl.when(cid == 1)
        def _():
            pltpu.sync_copy(rel_hi.at[pl.ds(o, ZROWS)], zbuf)

        pltpu.sync_copy(zbuf, rel_sh.at[pl.ds(o, ZROWS)])

    def _zrow(i, _):
        for j in range(H // 16):
            zbuf[i, pl.ds(j * 16, 16)] = jnp.zeros((16,), jnp.float32)
        return 0
    lax.fori_loop(0, ZROWS, _zrow, 0)
    for k in range(W // 16):
        ones_v[pl.ds(k * 16, 16)] = jnp.ones((16,), jnp.float32)
    if with_deg:
        def _zdeg(i, _):
            zdeg_v[pl.ds(i * 16, 16)] = jnp.zeros((16,), jnp.float32)
            return 0
        lax.fori_loop(0, 1024 // 16, _zdeg, 0)

    for k in range(ACC_SLAB // ZROWS):
        off = sid * ACC_SLAB + k * ZROWS

        @pl.when(off < NUM_ENT)
        def _():
            pltpu.sync_copy(zbuf,
                            acc_sh.at[pl.ds(pl.multiple_of(off, 8), ZROWS)])
    if with_deg:
        @pl.when((cid == 0) & (sid < 10))
        def _():
            pltpu.sync_copy(
                zdeg_v, deg_sh.at[pl.ds(pl.multiple_of(sid * 1024, 8), 1024)])

    plsc.subcore_barrier()

    def _window(i, _):
        base = pl.multiple_of(sid * EDGES_PER_TILE + i * W, 8)
        pltpu.sync_copy(src_h.at[pl.ds(base, W)], src_v)
        pltpu.sync_copy(dst_h.at[pl.ds(base, W)], dst_v)
        pltpu.sync_copy(et_h.at[pl.ds(base, W)], et_v)

        @pl.when(cid == 0)
        def _():
            pltpu.async_copy(x_lo.at[src_v], xbuf, sem).wait()

        @pl.when(cid == 1)
        def _():
            pltpu.async_copy(x_hi.at[src_v], xbuf, sem).wait()

        pltpu.async_copy(rel_sh.at[et_v], relbuf, sem).wait()

        def _edge(e, _):
            for j in range(H // 16):
                xv = xbuf[e, pl.ds(j * 16, 16)]
                rv = relbuf[e, pl.ds(j * 16, 16)]
                xbuf[e, pl.ds(j * 16, 16)] = xv * rv
            return 0
        lax.fori_loop(0, W, _edge, 0)

        pltpu.sync_copy(xbuf, acc_sh.at[dst_v], add=True)
        if with_deg:
            @pl.when(cid == 0)
            def _():
                pltpu.sync_copy(ones_v, deg_sh.at[dst_v], add=True)
        return 0

    lax.fori_loop(0, WINDOWS, _window, 0)

    plsc.subcore_barrier()

    def _copy_out(out_ref):
        for k in range(ACC_SLAB // ZROWS):
            off = sid * ACC_SLAB + k * ZROWS

            @pl.when(off < NUM_ENT)
            def _():
                o = pl.multiple_of(off, 8)
                pltpu.sync_copy(acc_sh.at[pl.ds(o, ZROWS)], zbuf)
                pltpu.sync_copy(zbuf, out_ref.at[pl.ds(o, ZROWS)])

    @pl.when(cid == 0)
    def _():
        _copy_out(out_lo)

    @pl.when(cid == 1)
    def _():
        _copy_out(out_hi)

    if with_deg:
        @pl.when((cid == 0) & (sid < 10))
        def _():
            off = pl.multiple_of(sid * 1000, 8)
            pltpu.sync_copy(deg_sh.at[pl.ds(off, 1000)],
                            zdeg_v.at[pl.ds(0, 1000)])
            pltpu.sync_copy(zdeg_v.at[pl.ds(0, 1000)],
                            deg_out.at[pl.ds(off, 1000)])


def _make_sc_agg(with_deg):
    out_type = [
        jax.ShapeDtypeStruct((NUM_ENT, H), jnp.float32),
        jax.ShapeDtypeStruct((NUM_ENT, H), jnp.float32),
    ]
    if with_deg:
        out_type.append(jax.ShapeDtypeStruct((NUM_ENT,), jnp.float32))
    mesh = plsc.VectorSubcoreMesh(core_axis_name="c", subcore_axis_name="s")
    return pl.kernel(
        functools.partial(_sc_agg_body, with_deg),
        out_type=out_type,
        mesh=mesh,
        scratch_types=[
            pltpu.VMEM((W, H), jnp.float32),
            pltpu.VMEM((W, H), jnp.float32),
            pltpu.VMEM((ZROWS, H), jnp.float32),
            pltpu.VMEM((W,), jnp.int32),
            pltpu.VMEM((W,), jnp.int32),
            pltpu.VMEM((W,), jnp.int32),
            pltpu.VMEM((W,), jnp.float32),
            pltpu.VMEM((1024,), jnp.float32),
            pltpu.VMEM_SHARED((NUM_RELROWS, H), jnp.float32),
            pltpu.VMEM_SHARED((NUM_ENT, H), jnp.float32),
            pltpu.VMEM_SHARED((DEG_PAD,), jnp.float32),
            pltpu.SemaphoreType.DMA,
        ],
        compiler_params=pltpu.CompilerParams(needs_layout_passes=False),
        name="sc_agg_deg" if with_deg else "sc_agg",
    )


_sc_agg_deg = _make_sc_agg(True)
_sc_agg = _make_sc_agg(False)


def _sc_lookup_body(x_h, r_h, sub_h, rel_h, sub_out, rel_out,
                    idx_v, xrows_v, rrows_v, sem):
    cid = lax.axis_index("c")
    sid = lax.axis_index("s")
    wid = sid * 2 + cid
    bpw = BATCH // 32
    base = pl.multiple_of(wid * bpw, 8)
    pltpu.sync_copy(sub_h.at[pl.ds(base, bpw)], idx_v)
    pltpu.async_copy(x_h.at[idx_v], xrows_v, sem).wait()
    pltpu.sync_copy(xrows_v, sub_out.at[pl.ds(base, bpw)])
    pltpu.sync_copy(rel_h.at[pl.ds(base, bpw)], idx_v)
    pltpu.async_copy(r_h.at[idx_v], rrows_v, sem).wait()
    pltpu.sync_copy(rrows_v, rel_out.at[pl.ds(base, bpw)])


_sc_lookup = pl.kernel(
    _sc_lookup_body,
    out_type=[
        jax.ShapeDtypeStruct((BATCH, D), jnp.float32),
        jax.ShapeDtypeStruct((BATCH, D), jnp.float32),
    ],
    mesh=plsc.VectorSubcoreMesh(core_axis_name="c", subcore_axis_name="s"),
    scratch_types=[
        pltpu.VMEM((BATCH // 32,), jnp.int32),
        pltpu.VMEM((BATCH // 32, D), jnp.float32),
        pltpu.VMEM((BATCH // 32, D), jnp.float32),
        pltpu.SemaphoreType.DMA,
    ],
    name="sc_lookup",
)


RB = 1000


def _tc1_body(agg_lo, agg_hi, deg, W1, r0, W1r,
              x1lo, x1hi, r1lo, r1hi):
    i = pl.program_id(0)
    norm = 1.0 / jnp.maximum(deg[...], 1.0)
    a_lo = agg_lo[...] * norm
    a_hi = agg_hi[...] * norm
    w = W1[...]
    h = (jnp.dot(a_lo, w[:H, :], preferred_element_type=jnp.float32)
         + jnp.dot(a_hi, w[H:, :], preferred_element_type=jnp.float32))
    x1 = jnp.tanh(h)
    x1lo[...] = x1[:, :H]
    x1hi[...] = x1[:, H:]

    @pl.when(i == 0)
    def _():
        r1 = jnp.dot(r0[...], W1r[...], preferred_element_type=jnp.float32)
        r1lo[...] = r1[:, :H]
        r1hi[...] = r1[:, H:]


_tc1 = pl.pallas_call(
    _tc1_body,
    grid=(NUM_ENT // RB,),
    in_specs=[
        pl.BlockSpec((RB, H), lambda i: (i, 0)),
        pl.BlockSpec((RB, H), lambda i: (i, 0)),
        pl.BlockSpec((RB, 1), lambda i: (i, 0)),
        pl.BlockSpec((D, D), lambda i: (0, 0)),
        pl.BlockSpec((NUM_RELROWS, D), lambda i: (0, 0)),
        pl.BlockSpec((D, D), lambda i: (0, 0)),
    ],
    out_specs=[
        pl.BlockSpec((RB, H), lambda i: (i, 0)),
        pl.BlockSpec((RB, H), lambda i: (i, 0)),
        pl.BlockSpec((NUM_RELROWS, H), lambda i: (0, 0)),
        pl.BlockSpec((NUM_RELROWS, H), lambda i: (0, 0)),
    ],
    out_shape=[
        jax.ShapeDtypeStruct((NUM_ENT, H), jnp.float32),
        jax.ShapeDtypeStruct((NUM_ENT, H), jnp.float32),
        jax.ShapeDtypeStruct((NUM_RELROWS, H), jnp.float32),
        jax.ShapeDtypeStruct((NUM_RELROWS, H), jnp.float32),
    ],
    name="tc_dense1",
)


def _tc2_body(agg_lo, agg_hi, deg, W2, r1lo, r1hi, W2r,
              x2, r2_out, cor_out):
    i = pl.program_id(0)
    norm = 1.0 / jnp.maximum(deg[...], 1.0)
    a_lo = agg_lo[...] * norm
    a_hi = agg_hi[...] * norm
    w = W2[...]
    h = (jnp.dot(a_lo, w[:H, :], preferred_element_type=jnp.float32)
         + jnp.dot(a_hi, w[H:, :], preferred_element_type=jnp.float32))
    x2[...] = jnp.tanh(h)

    @pl.when(i == 0)
    def _():
        wr = W2r[...]
        r2 = (jnp.dot(r1lo[...], wr[:H, :], preferred_element_type=jnp.float32)
              + jnp.dot(r1hi[...], wr[H:, :], preferred_element_type=jnp.float32))
        r2_out[...] = r2
        nrm = jnp.sqrt(jnp.sum(r2 * r2, axis=1, keepdims=True))
        nr = r2 / nrm
        pos_smi = jnp.sum(nr * nr, axis=1)
        ttl_smi = lax.dot_general(nr, nr, (((1,), (1,)), ((), ())),
                                  preferred_element_type=jnp.float32)
        pos_scores = jnp.exp(pos_smi / TEMPERATURE)
        ttl_scores = jnp.exp(ttl_smi / TEMPERATURE)
        semi = jnp.exp(jnp.float32(1.0) / TEMPERATURE)
        ttl = jnp.sum(ttl_scores, axis=1) - semi + pos_scores
        cor = -jnp.sum(jnp.log(pos_scores / ttl))
        cor_out[...] = cor.reshape(1, 1)


_tc2 = pl.pallas_call(
    _tc2_body,
    grid=(NUM_ENT // RB,),
    in_specs=[
        pl.BlockSpec((RB, H), lambda i: (i, 0)),
        pl.BlockSpec((RB, H), lambda i: (i, 0)),
        pl.BlockSpec((RB, 1), lambda i: (i, 0)),
        pl.BlockSpec((D, D), lambda i: (0, 0)),
        pl.BlockSpec((NUM_RELROWS, H), lambda i: (0, 0)),
        pl.BlockSpec((NUM_RELROWS, H), lambda i: (0, 0)),
        pl.BlockSpec((D, D), lambda i: (0, 0)),
    ],
    out_specs=[
        pl.BlockSpec((RB, D), lambda i: (i, 0)),
        pl.BlockSpec((NUM_RELROWS, D), lambda i: (0, 0)),
        pl.BlockSpec((1, 1), lambda i: (0, 0)),
    ],
    out_shape=[
        jax.ShapeDtypeStruct((NUM_ENT, D), jnp.float32),
        jax.ShapeDtypeStruct((NUM_RELROWS, D), jnp.float32),
        jax.ShapeDtypeStruct((1, 1), jnp.float32),
    ],
    name="tc_dense2",
)


def kernel(sub, rel, edge_index, edge_type, init_embed, init_rel,
           W1, W1_rel, W2, W2_rel):
    sub = sub.astype(jnp.int32)
    rel = rel.astype(jnp.int32)
    src = edge_index[0].astype(jnp.int32)
    dst = edge_index[1].astype(jnp.int32)
    et = edge_type.astype(jnp.int32)

    x0_lo = init_embed[:, :H]
    x0_hi = init_embed[:, H:]
    r0_lo = init_rel[:, :H]
    r0_hi = init_rel[:, H:]

    agg1_lo, agg1_hi, deg = _sc_agg_deg(x0_lo, x0_hi, r0_lo, r0_hi,
                                        src, dst, et)
    deg2d = deg.reshape(NUM_ENT, 1)
    x1lo, x1hi, r1lo, r1hi = _tc1(agg1_lo, agg1_hi, deg2d, W1,
                                  init_rel, W1_rel)
    agg2_lo, agg2_hi = _sc_agg(x1lo, x1hi, r1lo, r1hi, src, dst, et)
    x2, r2, cor = _tc2(agg2_lo, agg2_hi, deg2d, W2, r1lo, r1hi, W2_rel)
    sub_emb, rel_emb = _sc_lookup(x2, r2, sub, rel)
    return sub_emb, rel_emb, x2, cor[0, 0]


# speedup vs baseline: 3.1811x; 1.1608x over previous
import functools

import jax
import jax.numpy as jnp
from jax import lax
from jax.experimental import pallas as pl
from jax.experimental.pallas import tpu as pltpu
from jax.experimental.pallas import tpu_sc as plsc

NUM_ENT = 10000
NUM_RELROWS = 400
N_EDGES = 160000
D = 256
H = 128
NS = 16
W = 128
WINDOWS_TOTAL = N_EDGES // W
NITER = -(-WINDOWS_TOTAL // NS)
ACC_SLAB = 640
ZROWS = 80
DEG_PAD = 10240
TEMPERATURE = 0.5
BATCH = 1024


def _sc_agg_body(with_deg, x_lo, x_hi, rel_lo, rel_hi, src_h, dst_h, et_h,
                 *refs):
    if with_deg:
        out_lo, out_hi, deg_out = refs[:3]
        scratch = refs[3:]
    else:
        out_lo, out_hi = refs[:2]
        scratch = refs[2:]
    (xbuf, relbuf, zbuf, src_v, dst_v, et_v, ones_v,
     zdeg_v, rel_sh, acc_sh, deg_sh, sem) = scratch

    cid = lax.axis_index("c")
    sid = lax.axis_index("s")

    @pl.when(sid < NUM_RELROWS // ZROWS)
    def _():
        o = pl.multiple_of(sid * ZROWS, 8)

        @pl.when(cid == 0)
        def _():
            pltpu.sync_copy(rel_lo.at[pl.ds(o, ZROWS)], zbuf)

        @pl.when(cid == 1)
        def _():
            pltpu.sync_copy(rel_hi.at[pl.ds(o, ZROWS)], zbuf)

        pltpu.sync_copy(zbuf, rel_sh.at[pl.ds(o, ZROWS)])

    def _zrow(i, _):
        for j in range(H // 16):
            zbuf[i, pl.ds(j * 16, 16)] = jnp.zeros((16,), jnp.float32)
        return 0
    lax.fori_loop(0, ZROWS, _zrow, 0)
    for k in range(W // 16):
        ones_v[pl.ds(k * 16, 16)] = jnp.ones((16,), jnp.float32)
    if with_deg:
        def _zdeg(i, _):
            zdeg_v[pl.ds(i * 16, 16)] = jnp.zeros((16,), jnp.float32)
            return 0
        lax.fori_loop(0, 1024 // 16, _zdeg, 0)

    for k in range(ACC_SLAB // ZROWS):
        off = sid * ACC_SLAB + k * ZROWS

        @pl.when(off < NUM_ENT)
        def _():
            pltpu.sync_copy(zbuf,
                            acc_sh.at[pl.ds(pl.multiple_of(off, 8), ZROWS)])
    if with_deg:
        @pl.when((cid == 0) & (sid < 10))
        def _():
            pltpu.sync_copy(
                zdeg_v, deg_sh.at[pl.ds(pl.multiple_of(sid * 1024, 8), 1024)])

    plsc.subcore_barrier()

    def _window(i, _):
        widx = sid + i * NS

        @pl.when(widx < WINDOWS_TOTAL)
        def _():
            base = pl.multiple_of(widx * W, 8)
            pltpu.sync_copy(src_h.at[pl.ds(base, W)], src_v)
            pltpu.sync_copy(dst_h.at[pl.ds(base, W)], dst_v)
            pltpu.sync_copy(et_h.at[pl.ds(base, W)], et_v)

            @pl.when(cid == 0)
            def _():
                pltpu.async_copy(x_lo.at[src_v], xbuf, sem).wait()

            @pl.when(cid == 1)
            def _():
                pltpu.async_copy(x_hi.at[src_v], xbuf, sem).wait()

            pltpu.async_copy(rel_sh.at[et_v], relbuf, sem).wait()

            def _edge(e, _):
                for j in range(H // 16):
                    xv = xbuf[e, pl.ds(j * 16, 16)]
                    rv = relbuf[e, pl.ds(j * 16, 16)]
                    xbuf[e, pl.ds(j * 16, 16)] = xv * rv
                return 0
            lax.fori_loop(0, W, _edge, 0)

            pltpu.sync_copy(xbuf, acc_sh.at[dst_v], add=True)
            if with_deg:
                @pl.when(cid == 0)
                def _():
                    pltpu.sync_copy(ones_v, deg_sh.at[dst_v], add=True)
        return 0

    lax.fori_loop(0, NITER, _window, 0)

    plsc.subcore_barrier()

    def _copy_out(out_ref):
        for k in range(ACC_SLAB // ZROWS):
            off = sid * ACC_SLAB + k * ZROWS

            @pl.when(off < NUM_ENT)
            def _():
                o = pl.multiple_of(off, 8)
                pltpu.sync_copy(acc_sh.at[pl.ds(o, ZROWS)], zbuf)
                pltpu.sync_copy(zbuf, out_ref.at[pl.ds(o, ZROWS)])

    @pl.when(cid == 0)
    def _():
        _copy_out(out_lo)

    @pl.when(cid == 1)
    def _():
        _copy_out(out_hi)

    if with_deg:
        @pl.when((cid == 0) & (sid < 10))
        def _():
            off = pl.multiple_of(sid * 1000, 8)
            pltpu.sync_copy(deg_sh.at[pl.ds(off, 1000)],
                            zdeg_v.at[pl.ds(0, 1000)])
            pltpu.sync_copy(zdeg_v.at[pl.ds(0, 1000)],
                            deg_out.at[pl.ds(off, 1000)])


def _make_sc_agg(with_deg):
    out_type = [
        jax.ShapeDtypeStruct((NUM_ENT, H), jnp.float32),
        jax.ShapeDtypeStruct((NUM_ENT, H), jnp.float32),
    ]
    if with_deg:
        out_type.append(jax.ShapeDtypeStruct((NUM_ENT,), jnp.float32))
    mesh = plsc.VectorSubcoreMesh(core_axis_name="c", subcore_axis_name="s")
    return pl.kernel(
        functools.partial(_sc_agg_body, with_deg),
        out_type=out_type,
        mesh=mesh,
        scratch_types=[
            pltpu.VMEM((W, H), jnp.float32),
            pltpu.VMEM((W, H), jnp.float32),
            pltpu.VMEM((ZROWS, H), jnp.float32),
            pltpu.VMEM((W,), jnp.int32),
            pltpu.VMEM((W,), jnp.int32),
            pltpu.VMEM((W,), jnp.int32),
            pltpu.VMEM((W,), jnp.float32),
            pltpu.VMEM((1024,), jnp.float32),
            pltpu.VMEM_SHARED((NUM_RELROWS, H), jnp.float32),
            pltpu.VMEM_SHARED((NUM_ENT, H), jnp.float32),
            pltpu.VMEM_SHARED((DEG_PAD,), jnp.float32),
            pltpu.SemaphoreType.DMA,
        ],
        compiler_params=pltpu.CompilerParams(needs_layout_passes=False),
        name="sc_agg_deg" if with_deg else "sc_agg",
    )


_sc_agg_deg = _make_sc_agg(True)
_sc_agg = _make_sc_agg(False)


def _sc_lookup_body(x_h, r_h, sub_h, rel_h, sub_out, rel_out,
                    idx_v, xrows_v, rrows_v, sem):
    cid = lax.axis_index("c")
    sid = lax.axis_index("s")
    wid = sid * 2 + cid
    bpw = BATCH // 32
    base = pl.multiple_of(wid * bpw, 8)
    pltpu.sync_copy(sub_h.at[pl.ds(base, bpw)], idx_v)
    pltpu.async_copy(x_h.at[idx_v], xrows_v, sem).wait()
    pltpu.sync_copy(xrows_v, sub_out.at[pl.ds(base, bpw)])
    pltpu.sync_copy(rel_h.at[pl.ds(base, bpw)], idx_v)
    pltpu.async_copy(r_h.at[idx_v], rrows_v, sem).wait()
    pltpu.sync_copy(rrows_v, rel_out.at[pl.ds(base, bpw)])


_sc_lookup = pl.kernel(
    _sc_lookup_body,
    out_type=[
        jax.ShapeDtypeStruct((BATCH, D), jnp.float32),
        jax.ShapeDtypeStruct((BATCH, D), jnp.float32),
    ],
    mesh=plsc.VectorSubcoreMesh(core_axis_name="c", subcore_axis_name="s"),
    scratch_types=[
        pltpu.VMEM((BATCH // 32,), jnp.int32),
        pltpu.VMEM((BATCH // 32, D), jnp.float32),
        pltpu.VMEM((BATCH // 32, D), jnp.float32),
        pltpu.SemaphoreType.DMA,
    ],
    name="sc_lookup",
)


RB = 1000


def _tc1_body(agg_lo, agg_hi, deg, W1, r0, W1r,
              x1lo, x1hi, r1lo, r1hi):
    i = pl.program_id(0)
    norm = 1.0 / jnp.maximum(deg[...], 1.0)
    a_lo = agg_lo[...] * norm
    a_hi = agg_hi[...] * norm
    w = W1[...]
    h = (jnp.dot(a_lo, w[:H, :], preferred_element_type=jnp.float32)
         + jnp.dot(a_hi, w[H:, :], preferred_element_type=jnp.float32))
    x1 = jnp.tanh(h)
    x1lo[...] = x1[:, :H]
    x1hi[...] = x1[:, H:]

    @pl.when(i == 0)
    def _():
        r1 = jnp.dot(r0[...], W1r[...], preferred_element_type=jnp.float32)
        r1lo[...] = r1[:, :H]
        r1hi[...] = r1[:, H:]


_tc1 = pl.pallas_call(
    _tc1_body,
    grid=(NUM_ENT // RB,),
    in_specs=[
        pl.BlockSpec((RB, H), lambda i: (i, 0)),
        pl.BlockSpec((RB, H), lambda i: (i, 0)),
        pl.BlockSpec((RB, 1), lambda i: (i, 0)),
        pl.BlockSpec((D, D), lambda i: (0, 0)),
        pl.BlockSpec((NUM_RELROWS, D), lambda i: (0, 0)),
        pl.BlockSpec((D, D), lambda i: (0, 0)),
    ],
    out_specs=[
        pl.BlockSpec((RB, H), lambda i: (i, 0)),
        pl.BlockSpec((RB, H), lambda i: (i, 0)),
        pl.BlockSpec((NUM_RELROWS, H), lambda i: (0, 0)),
        pl.BlockSpec((NUM_RELROWS, H), lambda i: (0, 0)),
    ],
    out_shape=[
        jax.ShapeDtypeStruct((NUM_ENT, H), jnp.float32),
        jax.ShapeDtypeStruct((NUM_ENT, H), jnp.float32),
        jax.ShapeDtypeStruct((NUM_RELROWS, H), jnp.float32),
        jax.ShapeDtypeStruct((NUM_RELROWS, H), jnp.float32),
    ],
    name="tc_dense1",
)


def _tc2_body(agg_lo, agg_hi, deg, W2, r1lo, r1hi, W2r,
              x2, r2_out, cor_out):
    i = pl.program_id(0)
    norm = 1.0 / jnp.maximum(deg[...], 1.0)
    a_lo = agg_lo[...] * norm
    a_hi = agg_hi[...] * norm
    w = W2[...]
    h = (jnp.dot(a_lo, w[:H, :], preferred_element_type=jnp.float32)
         + jnp.dot(a_hi, w[H:, :], preferred_element_type=jnp.float32))
    x2[...] = jnp.tanh(h)

    @pl.when(i == 0)
    def _():
        wr = W2r[...]
        r2 = (jnp.dot(r1lo[...], wr[:H, :], preferred_element_type=jnp.float32)
              + jnp.dot(r1hi[...], wr[H:, :], preferred_element_type=jnp.float32))
        r2_out[...] = r2
        nrm = jnp.sqrt(jnp.sum(r2 * r2, axis=1, keepdims=True))
        nr = r2 / nrm
        pos_smi = jnp.sum(nr * nr, axis=1)
        ttl_smi = lax.dot_general(nr, nr, (((1,), (1,)), ((), ())),
                                  preferred_element_type=jnp.float32)
        pos_scores = jnp.exp(pos_smi / TEMPERATURE)
        ttl_scores = jnp.exp(ttl_smi / TEMPERATURE)
        semi = jnp.exp(jnp.float32(1.0) / TEMPERATURE)
        ttl = jnp.sum(ttl_scores, axis=1) - semi + pos_scores
        cor = -jnp.sum(jnp.log(pos_scores / ttl))
        cor_out[...] = cor.reshape(1, 1)


_tc2 = pl.pallas_call(
    _tc2_body,
    grid=(NUM_ENT // RB,),
    in_specs=[
        pl.BlockSpec((RB, H), lambda i: (i, 0)),
        pl.BlockSpec((RB, H), lambda i: (i, 0)),
        pl.BlockSpec((RB, 1), lambda i: (i, 0)),
        pl.BlockSpec((D, D), lambda i: (0, 0)),
        pl.BlockSpec((NUM_RELROWS, H), lambda i: (0, 0)),
        pl.BlockSpec((NUM_RELROWS, H), lambda i: (0, 0)),
        pl.BlockSpec((D, D), lambda i: (0, 0)),
    ],
    out_specs=[
        pl.BlockSpec((RB, D), lambda i: (i, 0)),
        pl.BlockSpec((NUM_RELROWS, D), lambda i: (0, 0)),
        pl.BlockSpec((1, 1), lambda i: (0, 0)),
    ],
    out_shape=[
        jax.ShapeDtypeStruct((NUM_ENT, D), jnp.float32),
        jax.ShapeDtypeStruct((NUM_RELROWS, D), jnp.float32),
        jax.ShapeDtypeStruct((1, 1), jnp.float32),
    ],
    name="tc_dense2",
)


def kernel(sub, rel, edge_index, edge_type, init_embed, init_rel,
           W1, W1_rel, W2, W2_rel):
    sub = sub.astype(jnp.int32)
    rel = rel.astype(jnp.int32)
    src = edge_index[0].astype(jnp.int32)
    dst = edge_index[1].astype(jnp.int32)
    et = edge_type.astype(jnp.int32)

    x0_lo = init_embed[:, :H]
    x0_hi = init_embed[:, H:]
    r0_lo = init_rel[:, :H]
    r0_hi = init_rel[:, H:]

    agg1_lo, agg1_hi, deg = _sc_agg_deg(x0_lo, x0_hi, r0_lo, r0_hi,
                                        src, dst, et)
    deg2d = deg.reshape(NUM_ENT, 1)
    x1lo, x1hi, r1lo, r1hi = _tc1(agg1_lo, agg1_hi, deg2d, W1,
                                  init_rel, W1_rel)
    agg2_lo, agg2_hi = _sc_agg(x1lo, x1hi, r1lo, r1hi, src, dst, et)
    x2, r2, cor = _tc2(agg2_lo, agg2_hi, deg2d, W2, r1lo, r1hi, W2_rel)
    sub_emb, rel_emb = _sc_lookup(x2, r2, sub, rel)
    return sub_emb, rel_emb, x2, cor[0, 0]


# speedup vs baseline: 3.6934x; 1.1610x over previous
import functools

import jax
import jax.numpy as jnp
from jax import lax
from jax.experimental import pallas as pl
from jax.experimental.pallas import tpu as pltpu
from jax.experimental.pallas import tpu_sc as plsc

NUM_ENT = 10000
NUM_RELROWS = 400
N_EDGES = 160000
D = 256
H = 128
NS = 16
W = 128
WINDOWS_TOTAL = N_EDGES // W
WPB = 8
NBAT = 10
ACC_SLAB = 640
ZROWS = 80
DEG_PAD = 10240
TEMPERATURE = 0.5
BATCH = 1024


def _sc_agg_body(with_deg, x_lo, x_hi, rel_lo, rel_hi, src_h, dst_h, et_h,
                 *refs):
    if with_deg:
        out_lo, out_hi, deg_out = refs[:3]
        scratch = refs[3:]
    else:
        out_lo, out_hi = refs[:2]
        scratch = refs[2:]
    (xbuf, relbuf, zbuf, src_b, et_b, dst_v, ones_v,
     zdeg_v, rel_sh, acc_sh, deg_sh, sem) = scratch

    cid = lax.axis_index("c")
    sid = lax.axis_index("s")

    @pl.when(sid < NUM_RELROWS // ZROWS)
    def _():
        o = pl.multiple_of(sid * ZROWS, 8)

        @pl.when(cid == 0)
        def _():
            pltpu.sync_copy(rel_lo.at[pl.ds(o, ZROWS)], zbuf)

        @pl.when(cid == 1)
        def _():
            pltpu.sync_copy(rel_hi.at[pl.ds(o, ZROWS)], zbuf)

        pltpu.sync_copy(zbuf, rel_sh.at[pl.ds(o, ZROWS)])

    def _zrow(i, _):
        for j in range(H // 16):
            zbuf[i, pl.ds(j * 16, 16)] = jnp.zeros((16,), jnp.float32)
        return 0
    lax.fori_loop(0, ZROWS, _zrow, 0)
    for k in range(W // 16):
        ones_v[pl.ds(k * 16, 16)] = jnp.ones((16,), jnp.float32)
    if with_deg:
        def _zdeg(i, _):
            zdeg_v[pl.ds(i * 16, 16)] = jnp.zeros((16,), jnp.float32)
            return 0
        lax.fori_loop(0, 1024 // 16, _zdeg, 0)

    for k in range(ACC_SLAB // ZROWS):
        off = sid * ACC_SLAB + k * ZROWS

        @pl.when(off < NUM_ENT)
        def _():
            pltpu.sync_copy(zbuf,
                            acc_sh.at[pl.ds(pl.multiple_of(off, 8), ZROWS)])
    if with_deg:
        @pl.when((cid == 0) & (sid < 10))
        def _():
            pltpu.sync_copy(
                zdeg_v, deg_sh.at[pl.ds(pl.multiple_of(sid * 1024, 8), 1024)])

    plsc.subcore_barrier()

    row0 = sid * 78 + jnp.minimum(sid, 2)
    nwin = jnp.where(sid < 2, 79, 78)

    def _bat(bat, _):
        b_off = pl.multiple_of((row0 + bat * WPB) * W, 8)
        pltpu.sync_copy(src_h.at[pl.ds(b_off, WPB * W)], src_b)
        pltpu.sync_copy(et_h.at[pl.ds(b_off, WPB * W)], et_b)
        for k in range(WPB):
            wloc = bat * WPB + k

            @pl.when(wloc < nwin)
            def _():
                base = pl.multiple_of((row0 + wloc) * W, 8)
                pltpu.sync_copy(dst_h.at[pl.ds(base, W)], dst_v)

                @pl.when(cid == 0)
                def _():
                    pltpu.async_copy(x_lo.at[src_b.at[pl.ds(k * W, W)]],
                                     xbuf, sem).wait()

                @pl.when(cid == 1)
                def _():
                    pltpu.async_copy(x_hi.at[src_b.at[pl.ds(k * W, W)]],
                                     xbuf, sem).wait()

                pltpu.async_copy(rel_sh.at[et_b.at[pl.ds(k * W, W)]],
                                 relbuf, sem).wait()

                def _edge(e, _):
                    for j in range(H // 16):
                        xv = xbuf[e, pl.ds(j * 16, 16)]
                        rv = relbuf[e, pl.ds(j * 16, 16)]
                        xbuf[e, pl.ds(j * 16, 16)] = xv * rv
                    return 0
                lax.fori_loop(0, W, _edge, 0)

                pltpu.sync_copy(xbuf, acc_sh.at[dst_v], add=True)
                if with_deg:
                    @pl.when(cid == 0)
                    def _():
                        pltpu.sync_copy(ones_v, deg_sh.at[dst_v], add=True)
        return 0

    lax.fori_loop(0, NBAT, _bat, 0)

    plsc.subcore_barrier()

    def _copy_out(out_ref):
        for k in range(ACC_SLAB // ZROWS):
            off = sid * ACC_SLAB + k * ZROWS

            @pl.when(off < NUM_ENT)
            def _():
                o = pl.multiple_of(off, 8)
                pltpu.sync_copy(acc_sh.at[pl.ds(o, ZROWS)], zbuf)
                pltpu.sync_copy(zbuf, out_ref.at[pl.ds(o, ZROWS)])

    @pl.when(cid == 0)
    def _():
        _copy_out(out_lo)

    @pl.when(cid == 1)
    def _():
        _copy_out(out_hi)

    if with_deg:
        @pl.when((cid == 0) & (sid < 10))
        def _():
            off = pl.multiple_of(sid * 1000, 8)
            pltpu.sync_copy(deg_sh.at[pl.ds(off, 1000)],
                            zdeg_v.at[pl.ds(0, 1000)])
            pltpu.sync_copy(zdeg_v.at[pl.ds(0, 1000)],
                            deg_out.at[pl.ds(off, 1000)])


def _make_sc_agg(with_deg):
    out_type = [
        jax.ShapeDtypeStruct((NUM_ENT, H), jnp.float32),
        jax.ShapeDtypeStruct((NUM_ENT, H), jnp.float32),
    ]
    if with_deg:
        out_type.append(jax.ShapeDtypeStruct((NUM_ENT,), jnp.float32))
    mesh = plsc.VectorSubcoreMesh(core_axis_name="c", subcore_axis_name="s")
    return pl.kernel(
        functools.partial(_sc_agg_body, with_deg),
        out_type=out_type,
        mesh=mesh,
        scratch_types=[
            pltpu.VMEM((W, H), jnp.float32),
            pltpu.VMEM((W, H), jnp.float32),
            pltpu.VMEM((ZROWS, H), jnp.float32),
            pltpu.VMEM((WPB * W,), jnp.int32),
            pltpu.VMEM((WPB * W,), jnp.int32),
            pltpu.VMEM((W,), jnp.int32),
            pltpu.VMEM((W,), jnp.float32),
            pltpu.VMEM((1024,), jnp.float32),
            pltpu.VMEM_SHARED((NUM_RELROWS, H), jnp.float32),
            pltpu.VMEM_SHARED((NUM_ENT, H), jnp.float32),
            pltpu.VMEM_SHARED((DEG_PAD,), jnp.float32),
            pltpu.SemaphoreType.DMA,
        ],
        compiler_params=pltpu.CompilerParams(needs_layout_passes=False),
        name="sc_agg_deg" if with_deg else "sc_agg",
    )


_sc_agg_deg = _make_sc_agg(True)
_sc_agg = _make_sc_agg(False)


def _sc_lookup_body(x_h, r_h, sub_h, rel_h, sub_out, rel_out,
                    idx_v, xrows_v, rrows_v, sem):
    cid = lax.axis_index("c")
    sid = lax.axis_index("s")
    wid = sid * 2 + cid
    bpw = BATCH // 32
    base = pl.multiple_of(wid * bpw, 8)
    pltpu.sync_copy(sub_h.at[pl.ds(base, bpw)], idx_v)
    pltpu.async_copy(x_h.at[idx_v], xrows_v, sem).wait()
    pltpu.sync_copy(xrows_v, sub_out.at[pl.ds(base, bpw)])
    pltpu.sync_copy(rel_h.at[pl.ds(base, bpw)], idx_v)
    pltpu.async_copy(r_h.at[idx_v], rrows_v, sem).wait()
    pltpu.sync_copy(rrows_v, rel_out.at[pl.ds(base, bpw)])


_sc_lookup = pl.kernel(
    _sc_lookup_body,
    out_type=[
        jax.ShapeDtypeStruct((BATCH, D), jnp.float32),
        jax.ShapeDtypeStruct((BATCH, D), jnp.float32),
    ],
    mesh=plsc.VectorSubcoreMesh(core_axis_name="c", subcore_axis_name="s"),
    scratch_types=[
        pltpu.VMEM((BATCH // 32,), jnp.int32),
        pltpu.VMEM((BATCH // 32, D), jnp.float32),
        pltpu.VMEM((BATCH // 32, D), jnp.float32),
        pltpu.SemaphoreType.DMA,
    ],
    name="sc_lookup",
)


RB = 1000


def _tc1_body(agg_lo, agg_hi, deg, W1, r0, W1r,
              x1lo, x1hi, r1lo, r1hi):
    i = pl.program_id(0)
    norm = 1.0 / jnp.maximum(deg[...], 1.0)
    a_lo = agg_lo[...] * norm
    a_hi = agg_hi[...] * norm
    w = W1[...]
    h = (jnp.dot(a_lo, w[:H, :], preferred_element_type=jnp.float32)
         + jnp.dot(a_hi, w[H:, :], preferred_element_type=jnp.float32))
    x1 = jnp.tanh(h)
    x1lo[...] = x1[:, :H]
    x1hi[...] = x1[:, H:]

    @pl.when(i == 0)
    def _():
        r1 = jnp.dot(r0[...], W1r[...], preferred_element_type=jnp.float32)
        r1lo[...] = r1[:, :H]
        r1hi[...] = r1[:, H:]


_tc1 = pl.pallas_call(
    _tc1_body,
    grid=(NUM_ENT // RB,),
    in_specs=[
        pl.BlockSpec((RB, H), lambda i: (i, 0)),
        pl.BlockSpec((RB, H), lambda i: (i, 0)),
        pl.BlockSpec((RB, 1), lambda i: (i, 0)),
        pl.BlockSpec((D, D), lambda i: (0, 0)),
        pl.BlockSpec((NUM_RELROWS, D), lambda i: (0, 0)),
        pl.BlockSpec((D, D), lambda i: (0, 0)),
    ],
    out_specs=[
        pl.BlockSpec((RB, H), lambda i: (i, 0)),
        pl.BlockSpec((RB, H), lambda i: (i, 0)),
        pl.BlockSpec((NUM_RELROWS, H), lambda i: (0, 0)),
        pl.BlockSpec((NUM_RELROWS, H), lambda i: (0, 0)),
    ],
    out_shape=[
        jax.ShapeDtypeStruct((NUM_ENT, H), jnp.float32),
        jax.ShapeDtypeStruct((NUM_ENT, H), jnp.float32),
        jax.ShapeDtypeStruct((NUM_RELROWS, H), jnp.float32),
        jax.ShapeDtypeStruct((NUM_RELROWS, H), jnp.float32),
    ],
    name="tc_dense1",
)


def _tc2_body(agg_lo, agg_hi, deg, W2, r1lo, r1hi, W2r,
              x2, r2_out, cor_out):
    i = pl.program_id(0)
    norm = 1.0 / jnp.maximum(deg[...], 1.0)
    a_lo = agg_lo[...] * norm
    a_hi = agg_hi[...] * norm
    w = W2[...]
    h = (jnp.dot(a_lo, w[:H, :], preferred_element_type=jnp.float32)
         + jnp.dot(a_hi, w[H:, :], preferred_element_type=jnp.float32))
    x2[...] = jnp.tanh(h)

    @pl.when(i == 0)
    def _():
        wr = W2r[...]
        r2 = (jnp.dot(r1lo[...], wr[:H, :], preferred_element_type=jnp.float32)
              + jnp.dot(r1hi[...], wr[H:, :], preferred_element_type=jnp.float32))
        r2_out[...] = r2
        nrm = jnp.sqrt(jnp.sum(r2 * r2, axis=1, keepdims=True))
        nr = r2 / nrm
        pos_smi = jnp.sum(nr * nr, axis=1)
        ttl_smi = lax.dot_general(nr, nr, (((1,), (1,)), ((), ())),
                                  preferred_element_type=jnp.float32)
        pos_scores = jnp.exp(pos_smi / TEMPERATURE)
        ttl_scores = jnp.exp(ttl_smi / TEMPERATURE)
        semi = jnp.exp(jnp.float32(1.0) / TEMPERATURE)
        ttl = jnp.sum(ttl_scores, axis=1) - semi + pos_scores
        cor = -jnp.sum(jnp.log(pos_scores / ttl))
        cor_out[...] = cor.reshape(1, 1)


_tc2 = pl.pallas_call(
    _tc2_body,
    grid=(NUM_ENT // RB,),
    in_specs=[
        pl.BlockSpec((RB, H), lambda i: (i, 0)),
        pl.BlockSpec((RB, H), lambda i: (i, 0)),
        pl.BlockSpec((RB, 1), lambda i: (i, 0)),
        pl.BlockSpec((D, D), lambda i: (0, 0)),
        pl.BlockSpec((NUM_RELROWS, H), lambda i: (0, 0)),
        pl.BlockSpec((NUM_RELROWS, H), lambda i: (0, 0)),
        pl.BlockSpec((D, D), lambda i: (0, 0)),
    ],
    out_specs=[
        pl.BlockSpec((RB, D), lambda i: (i, 0)),
        pl.BlockSpec((NUM_RELROWS, D), lambda i: (0, 0)),
        pl.BlockSpec((1, 1), lambda i: (0, 0)),
    ],
    out_shape=[
        jax.ShapeDtypeStruct((NUM_ENT, D), jnp.float32),
        jax.ShapeDtypeStruct((NUM_RELROWS, D), jnp.float32),
        jax.ShapeDtypeStruct((1, 1), jnp.float32),
    ],
    name="tc_dense2",
)


def kernel(sub, rel, edge_index, edge_type, init_embed, init_rel,
           W1, W1_rel, W2, W2_rel):
    sub = sub.astype(jnp.int32)
    rel = rel.astype(jnp.int32)
    pad = jnp.zeros((WPB * W,), jnp.int32)
    src = jnp.concatenate([edge_index[0].astype(jnp.int32), pad])
    dst = jnp.concatenate([edge_index[1].astype(jnp.int32), pad])
    et = jnp.concatenate([edge_type.astype(jnp.int32), pad])

    x0_lo = init_embed[:, :H]
    x0_hi = init_embed[:, H:]
    r0_lo = init_rel[:, :H]
    r0_hi = init_rel[:, H:]

    agg1_lo, agg1_hi, deg = _sc_agg_deg(x0_lo, x0_hi, r0_lo, r0_hi,
                                        src, dst, et)
    deg2d = deg.reshape(NUM_ENT, 1)
    x1lo, x1hi, r1lo, r1hi = _tc1(agg1_lo, agg1_hi, deg2d, W1,
                                  init_rel, W1_rel)
    agg2_lo, agg2_hi = _sc_agg(x1lo, x1hi, r1lo, r1hi, src, dst, et)
    x2, r2, cor = _tc2(agg2_lo, agg2_hi, deg2d, W2, r1lo, r1hi, W2_rel)
    sub_emb, rel_emb = _sc_lookup(x2, r2, sub, rel)
    return sub_emb, rel_emb, x2, cor[0, 0]


# speedup vs baseline: 3.7870x; 1.0253x over previous
import functools

import jax
import jax.numpy as jnp
from jax import lax
from jax.experimental import pallas as pl
from jax.experimental.pallas import tpu as pltpu
from jax.experimental.pallas import tpu_sc as plsc

NUM_ENT = 10000
NUM_RELROWS = 400
N_EDGES = 160000
D = 256
H = 128
NS = 16
W = 128
WINDOWS_TOTAL = N_EDGES // W
WPB = 8
NBAT = 10
ROWS_PAD = 16 * 80
ACC_SLAB = 640
ZROWS = 80
DEG_PAD = 10240
TEMPERATURE = 0.5
BATCH = 1024


def _sc_agg_body(with_deg, x_lo, x_hi, rel_lo, rel_hi, src_h, et_h, dst2_h,
                 *refs):
    if with_deg:
        out_lo, out_hi, deg_out = refs[:3]
        scratch = refs[3:]
    else:
        out_lo, out_hi = refs[:2]
        scratch = refs[2:]
    (xbuf, relbuf, src_b, et_b, dst_b, ones_v,
     zdeg_v, rel_sh, acc_sh, deg_sh, sem) = scratch

    cid = lax.axis_index("c")
    sid = lax.axis_index("s")

    @pl.when(sid < 3)
    def _():
        o = pl.multiple_of(sid * 128, 8)
        pltpu.sync_copy(rel_lo.at[pl.ds(o, 128)], xbuf)

        @pl.when(cid == 1)
        def _():
            pltpu.sync_copy(rel_hi.at[pl.ds(o, 128)], xbuf)

        pltpu.sync_copy(xbuf, rel_sh.at[pl.ds(o, 128)])

    @pl.when(sid == 3)
    def _():
        pltpu.sync_copy(rel_lo.at[pl.ds(384, 16)], xbuf.at[pl.ds(0, 16)])

        @pl.when(cid == 1)
        def _():
            pltpu.sync_copy(rel_hi.at[pl.ds(384, 16)], xbuf.at[pl.ds(0, 16)])

        pltpu.sync_copy(xbuf.at[pl.ds(0, 16)], rel_sh.at[pl.ds(384, 16)])

    def _zrow(i, _):
        for j in range(H // 16):
            xbuf[i, pl.ds(j * 16, 16)] = jnp.zeros((16,), jnp.float32)
        return 0
    lax.fori_loop(0, W, _zrow, 0)
    for k in range(W // 16):
        ones_v[pl.ds(k * 16, 16)] = jnp.ones((16,), jnp.float32)
    if with_deg:
        def _zdeg(i, _):
            zdeg_v[pl.ds(i * 16, 16)] = jnp.zeros((16,), jnp.float32)
            return 0
        lax.fori_loop(0, 1024 // 16, _zdeg, 0)

    for k in range(ACC_SLAB // W):
        off = sid * ACC_SLAB + k * W

        @pl.when(off + W <= NUM_ENT)
        def _():
            pltpu.sync_copy(xbuf,
                            acc_sh.at[pl.ds(pl.multiple_of(off, 8), W)])

    @pl.when(sid == 15)
    def _():
        pltpu.sync_copy(xbuf.at[pl.ds(0, 16)],
                        acc_sh.at[pl.ds(NUM_ENT - 16, 16)])

    if with_deg:
        @pl.when((cid == 0) & (sid < 10))
        def _():
            pltpu.sync_copy(
                zdeg_v, deg_sh.at[pl.ds(pl.multiple_of(sid * 1024, 8), 1024)])

    plsc.subcore_barrier()

    row0 = sid * 80
    nwin = jnp.where(sid < 15, 80, 50)

    def _bat(bat, _):
        b_off = pl.multiple_of((row0 + bat * WPB) * W, 8)
        pltpu.sync_copy(src_h.at[pl.ds(b_off, WPB * W)], src_b)
        pltpu.sync_copy(et_h.at[pl.ds(b_off, WPB * W)], et_b)
        brow = pl.multiple_of(row0 + bat * WPB, 8)
        pltpu.sync_copy(dst2_h.at[pl.ds(brow, WPB)], dst_b)
        for k in range(WPB):
            wloc = bat * WPB + k

            @pl.when(wloc < nwin)
            def _():
                @pl.when(cid == 0)
                def _():
                    pltpu.async_copy(x_lo.at[src_b.at[pl.ds(k * W, W)]],
                                     xbuf, sem).wait()

                @pl.when(cid == 1)
                def _():
                    pltpu.async_copy(x_hi.at[src_b.at[pl.ds(k * W, W)]],
                                     xbuf, sem).wait()

                pltpu.async_copy(rel_sh.at[et_b.at[pl.ds(k * W, W)]],
                                 relbuf, sem).wait()

                def _edge(e, _):
                    for j in range(H // 16):
                        xv = xbuf[e, pl.ds(j * 16, 16)]
                        rv = relbuf[e, pl.ds(j * 16, 16)]
                        xbuf[e, pl.ds(j * 16, 16)] = xv * rv
                    return 0
                lax.fori_loop(0, W, _edge, 0)

                pltpu.sync_copy(xbuf, acc_sh.at[dst_b.at[k]], add=True)
                if with_deg:
                    @pl.when(cid == 0)
                    def _():
                        pltpu.sync_copy(ones_v, deg_sh.at[dst_b.at[k]],
                                        add=True)
        return 0

    lax.fori_loop(0, NBAT, _bat, 0)

    plsc.subcore_barrier()

    def _copy_out(out_ref):
        for k in range(ACC_SLAB // W):
            off = sid * ACC_SLAB + k * W

            @pl.when(off + W <= NUM_ENT)
            def _():
                o = pl.multiple_of(off, 8)
                pltpu.sync_copy(acc_sh.at[pl.ds(o, W)], xbuf)
                pltpu.sync_copy(xbuf, out_ref.at[pl.ds(o, W)])

        @pl.when(sid == 15)
        def _():
            pltpu.sync_copy(acc_sh.at[pl.ds(NUM_ENT - 16, 16)],
                            xbuf.at[pl.ds(0, 16)])
            pltpu.sync_copy(xbuf.at[pl.ds(0, 16)],
                            out_ref.at[pl.ds(NUM_ENT - 16, 16)])

    @pl.when(cid == 0)
    def _():
        _copy_out(out_lo)

    @pl.when(cid == 1)
    def _():
        _copy_out(out_hi)

    if with_deg:
        @pl.when((cid == 0) & (sid < 10))
        def _():
            off = pl.multiple_of(sid * 1000, 8)
            pltpu.sync_copy(deg_sh.at[pl.ds(off, 1000)],
                            zdeg_v.at[pl.ds(0, 1000)])
            pltpu.sync_copy(zdeg_v.at[pl.ds(0, 1000)],
                            deg_out.at[pl.ds(off, 1000)])


def _make_sc_agg(with_deg):
    out_type = [
        jax.ShapeDtypeStruct((NUM_ENT, H), jnp.float32),
        jax.ShapeDtypeStruct((NUM_ENT, H), jnp.float32),
    ]
    if with_deg:
        out_type.append(jax.ShapeDtypeStruct((NUM_ENT,), jnp.float32))
    mesh = plsc.VectorSubcoreMesh(core_axis_name="c", subcore_axis_name="s")
    return pl.kernel(
        functools.partial(_sc_agg_body, with_deg),
        out_type=out_type,
        mesh=mesh,
        scratch_types=[
            pltpu.VMEM((W, H), jnp.float32),
            pltpu.VMEM((W, H), jnp.float32),
            pltpu.VMEM((WPB * W,), jnp.int32),
            pltpu.VMEM((WPB * W,), jnp.int32),
            pltpu.VMEM((WPB, W), jnp.int32),
            pltpu.VMEM((W,), jnp.float32),
            pltpu.VMEM((1024,), jnp.float32),
            pltpu.VMEM_SHARED((NUM_RELROWS, H), jnp.float32),
            pltpu.VMEM_SHARED((NUM_ENT, H), jnp.float32),
            pltpu.VMEM_SHARED((DEG_PAD,), jnp.float32),
            pltpu.SemaphoreType.DMA,
        ],
        compiler_params=pltpu.CompilerParams(needs_layout_passes=False),
        name="sc_agg_deg" if with_deg else "sc_agg",
    )


_sc_agg_deg = _make_sc_agg(True)
_sc_agg = _make_sc_agg(False)


def _sc_lookup_body(x_h, r_h, sub_h, rel_h, sub_out, rel_out,
                    idx_v, xrows_v, rrows_v, sem):
    cid = lax.axis_index("c")
    sid = lax.axis_index("s")
    wid = sid * 2 + cid
    bpw = BATCH // 32
    base = pl.multiple_of(wid * bpw, 8)
    pltpu.sync_copy(sub_h.at[pl.ds(base, bpw)], idx_v)
    pltpu.async_copy(x_h.at[idx_v], xrows_v, sem).wait()
    pltpu.sync_copy(xrows_v, sub_out.at[pl.ds(base, bpw)])
    pltpu.sync_copy(rel_h.at[pl.ds(base, bpw)], idx_v)
    pltpu.async_copy(r_h.at[idx_v], rrows_v, sem).wait()
    pltpu.sync_copy(rrows_v, rel_out.at[pl.ds(base, bpw)])


_sc_lookup = pl.kernel(
    _sc_lookup_body,
    out_type=[
        jax.ShapeDtypeStruct((BATCH, D), jnp.float32),
        jax.ShapeDtypeStruct((BATCH, D), jnp.float32),
    ],
    mesh=plsc.VectorSubcoreMesh(core_axis_name="c", subcore_axis_name="s"),
    scratch_types=[
        pltpu.VMEM((BATCH // 32,), jnp.int32),
        pltpu.VMEM((BATCH // 32, D), jnp.float32),
        pltpu.VMEM((BATCH // 32, D), jnp.float32),
        pltpu.SemaphoreType.DMA,
    ],
    name="sc_lookup",
)


RB = 1000


def _tc1_body(agg_lo, agg_hi, deg, W1, r0, W1r,
              x1lo, x1hi, r1lo, r1hi):
    i = pl.program_id(0)
    norm = 1.0 / jnp.maximum(deg[...], 1.0)
    a_lo = agg_lo[...] * norm
    a_hi = agg_hi[...] * norm
    w = W1[...]
    h = (jnp.dot(a_lo, w[:H, :], preferred_element_type=jnp.float32)
         + jnp.dot(a_hi, w[H:, :], preferred_element_type=jnp.float32))
    x1 = jnp.tanh(h)
    x1lo[...] = x1[:, :H]
    x1hi[...] = x1[:, H:]

    @pl.when(i == 0)
    def _():
        r1 = jnp.dot(r0[...], W1r[...], preferred_element_type=jnp.float32)
        r1lo[...] = r1[:, :H]
        r1hi[...] = r1[:, H:]


_tc1 = pl.pallas_call(
    _tc1_body,
    grid=(NUM_ENT // RB,),
    in_specs=[
        pl.BlockSpec((RB, H), lambda i: (i, 0)),
        pl.BlockSpec((RB, H), lambda i: (i, 0)),
        pl.BlockSpec((RB, 1), lambda i: (i, 0)),
        pl.BlockSpec((D, D), lambda i: (0, 0)),
        pl.BlockSpec((NUM_RELROWS, D), lambda i: (0, 0)),
        pl.BlockSpec((D, D), lambda i: (0, 0)),
    ],
    out_specs=[
        pl.BlockSpec((RB, H), lambda i: (i, 0)),
        pl.BlockSpec((RB, H), lambda i: (i, 0)),
        pl.BlockSpec((NUM_RELROWS, H), lambda i: (0, 0)),
        pl.BlockSpec((NUM_RELROWS, H), lambda i: (0, 0)),
    ],
    out_shape=[
        jax.ShapeDtypeStruct((NUM_ENT, H), jnp.float32),
        jax.ShapeDtypeStruct((NUM_ENT, H), jnp.float32),
        jax.ShapeDtypeStruct((NUM_RELROWS, H), jnp.float32),
        jax.ShapeDtypeStruct((NUM_RELROWS, H), jnp.float32),
    ],
    name="tc_dense1",
)


def _tc2_body(agg_lo, agg_hi, deg, W2, r1lo, r1hi, W2r,
              x2, r2_out, cor_out):
    i = pl.program_id(0)
    norm = 1.0 / jnp.maximum(deg[...], 1.0)
    a_lo = agg_lo[...] * norm
    a_hi = agg_hi[...] * norm
    w = W2[...]
    h = (jnp.dot(a_lo, w[:H, :], preferred_element_type=jnp.float32)
         + jnp.dot(a_hi, w[H:, :], preferred_element_type=jnp.float32))
    x2[...] = jnp.tanh(h)

    @pl.when(i == 0)
    def _():
        wr = W2r[...]
        r2 = (jnp.dot(r1lo[...], wr[:H, :], preferred_element_type=jnp.float32)
              + jnp.dot(r1hi[...], wr[H:, :], preferred_element_type=jnp.float32))
        r2_out[...] = r2
        nrm = jnp.sqrt(jnp.sum(r2 * r2, axis=1, keepdims=True))
        nr = r2 / nrm
        pos_smi = jnp.sum(nr * nr, axis=1)
        ttl_smi = lax.dot_general(nr, nr, (((1,), (1,)), ((), ())),
                                  preferred_element_type=jnp.float32)
        pos_scores = jnp.exp(pos_smi / TEMPERATURE)
        ttl_scores = jnp.exp(ttl_smi / TEMPERATURE)
        semi = jnp.exp(jnp.float32(1.0) / TEMPERATURE)
        ttl = jnp.sum(ttl_scores, axis=1) - semi + pos_scores
        cor = -jnp.sum(jnp.log(pos_scores / ttl))
        cor_out[...] = cor.reshape(1, 1)


_tc2 = pl.pallas_call(
    _tc2_body,
    grid=(NUM_ENT // RB,),
    in_specs=[
        pl.BlockSpec((RB, H), lambda i: (i, 0)),
        pl.BlockSpec((RB, H), lambda i: (i, 0)),
        pl.BlockSpec((RB, 1), lambda i: (i, 0)),
        pl.BlockSpec((D, D), lambda i: (0, 0)),
        pl.BlockSpec((NUM_RELROWS, H), lambda i: (0, 0)),
        pl.BlockSpec((NUM_RELROWS, H), lambda i: (0, 0)),
        pl.BlockSpec((D, D), lambda i: (0, 0)),
    ],
    out_specs=[
        pl.BlockSpec((RB, D), lambda i: (i, 0)),
        pl.BlockSpec((NUM_RELROWS, D), lambda i: (0, 0)),
        pl.BlockSpec((1, 1), lambda i: (0, 0)),
    ],
    out_shape=[
        jax.ShapeDtypeStruct((NUM_ENT, D), jnp.float32),
        jax.ShapeDtypeStruct((NUM_RELROWS, D), jnp.float32),
        jax.ShapeDtypeStruct((1, 1), jnp.float32),
    ],
    name="tc_dense2",
)


def kernel(sub, rel, edge_index, edge_type, init_embed, init_rel,
           W1, W1_rel, W2, W2_rel):
    sub = sub.astype(jnp.int32)
    rel = rel.astype(jnp.int32)
    pad = jnp.zeros((ROWS_PAD * W - N_EDGES,), jnp.int32)
    src = jnp.concatenate([edge_index[0].astype(jnp.int32), pad])
    dst2 = jnp.concatenate([edge_index[1].astype(jnp.int32), pad]).reshape(
        ROWS_PAD, W)
    et = jnp.concatenate([edge_type.astype(jnp.int32), pad])

    x0_lo = init_embed[:, :H]
    x0_hi = init_embed[:, H:]
    r0_lo = init_rel[:, :H]
    r0_hi = init_rel[:, H:]

    agg1_lo, agg1_hi, deg = _sc_agg_deg(x0_lo, x0_hi, r0_lo, r0_hi,
                                        src, et, dst2)
    deg2d = deg.reshape(NUM_ENT, 1)
    x1lo, x1hi, r1lo, r1hi = _tc1(agg1_lo, agg1_hi, deg2d, W1,
                                  init_rel, W1_rel)
    agg2_lo, agg2_hi = _sc_agg(x1lo, x1hi, r1lo, r1hi, src, et, dst2)
    x2, r2, cor = _tc2(agg2_lo, agg2_hi, deg2d, W2, r1lo, r1hi, W2_rel)
    sub_emb, rel_emb = _sc_lookup(x2, r2, sub, rel)
    return sub_emb, rel_emb, x2, cor[0, 0]
